# Initial kernel scaffold; baseline (speedup 1.0000x reference)
import functools
import jax
import jax.numpy as jnp
from jax import lax
from jax.experimental import pallas as pl
from jax.experimental.pallas import tpu as pltpu
from jax.experimental.pallas import tpu_sc as plsc

BATCH = 16384
VOCAB = 1000
D_USER = 32
D_DOW = 7
D_HOD = 4
D_OUT = D_USER + D_DOW + D_HOD

NC = 2
NS = 16
NW = NC * NS
L = 16
B_PER_W = BATCH // NW
GROUPS = B_PER_W // L


def _sc_kernel(user_id_hbm, dow_hbm, hod_hbm, utab_hbm, htab_hbm, out_hbm,
               uid_v, dow_v, hod_v, utab_v, htab_v, outbuf_v):
    wid = lax.axis_index("s") * NC + lax.axis_index("c")
    base = wid * B_PER_W

    pltpu.sync_copy(user_id_hbm.at[pl.ds(base, B_PER_W)], uid_v)
    pltpu.sync_copy(dow_hbm.at[pl.ds(base, B_PER_W)], dow_v)
    pltpu.sync_copy(hod_hbm.at[pl.ds(base, B_PER_W)], hod_v)
    pltpu.sync_copy(utab_hbm, utab_v)
    pltpu.sync_copy(htab_hbm, htab_v)

    lane = lax.iota(jnp.int32, L)

    def body(g, _):
        rows = g * L + lane
        uidx = uid_v[pl.ds(g * L, L)] + 1
        for j in range(D_USER):
            col = jnp.full((L,), j, jnp.int32)
            vals = plsc.load_gather(utab_v, [uidx, col])
            plsc.store_scatter(outbuf_v, [rows, col], vals)
        d = dow_v[pl.ds(g * L, L)]
        for c in range(D_DOW):
            v = jnp.where(d == c, 1.0, 0.0).astype(jnp.float32)
            col = jnp.full((L,), D_USER + c, jnp.int32)
            plsc.store_scatter(outbuf_v, [rows, col], v)
        hidx = jnp.minimum(hod_v[pl.ds(g * L, L)] + 1, 23)
        for j in range(D_HOD):
            col = jnp.full((L,), j, jnp.int32)
            vals = plsc.load_gather(htab_v, [hidx, col])
            ocol = jnp.full((L,), D_USER + D_DOW + j, jnp.int32)
            plsc.store_scatter(outbuf_v, [rows, ocol], vals)
        return 0

    lax.fori_loop(0, GROUPS, body, 0)

    pltpu.sync_copy(outbuf_v, out_hbm.at[pl.ds(base, B_PER_W)])


@jax.jit
def kernel(user_id, dow, hod, user_table, hod_table):
    mesh = plsc.VectorSubcoreMesh(core_axis_name="c", subcore_axis_name="s")
    run = functools.partial(
        pl.kernel, mesh=mesh,
        out_type=jax.ShapeDtypeStruct((BATCH, D_OUT), jnp.float32),
        scratch_types=[
            pltpu.VMEM((B_PER_W,), jnp.int32),
            pltpu.VMEM((B_PER_W,), jnp.int32),
            pltpu.VMEM((B_PER_W,), jnp.int32),
            pltpu.VMEM((VOCAB + 1, D_USER), jnp.float32),
            pltpu.VMEM((24, D_HOD), jnp.float32),
            pltpu.VMEM((B_PER_W, D_OUT), jnp.float32),
        ],
    )(_sc_kernel)
    return run(user_id, dow, hod, user_table, hod_table)


# speedup vs baseline: 2.2638x; 2.2638x over previous
import functools
import jax
import jax.numpy as jnp
from jax import lax
from jax.experimental import pallas as pl
from jax.experimental.pallas import tpu as pltpu
from jax.experimental.pallas import tpu_sc as plsc

BATCH = 16384
VOCAB = 1000
D_USER = 32
D_DOW = 7
D_HOD = 4
D_OUT = D_USER + D_DOW + D_HOD

NC = 2
NS = 16
NW = NC * NS
L = 16
B_PER_W = BATCH // NW
GROUPS = B_PER_W // L


def _sc_kernel(user_id_hbm, dow_hbm, hod_hbm, utab_hbm, htab_hbm, out_hbm,
               uid_v, dow_v, hod_v, utab_v, htab_v, outbuf_v):
    wid = lax.axis_index("s") * NC + lax.axis_index("c")
    base = wid * B_PER_W

    pltpu.sync_copy(user_id_hbm.at[pl.ds(base, B_PER_W)], uid_v)
    pltpu.sync_copy(dow_hbm.at[pl.ds(base, B_PER_W)], dow_v)
    pltpu.sync_copy(hod_hbm.at[pl.ds(base, B_PER_W)], hod_v)
    pltpu.sync_copy(utab_hbm, utab_v)
    pltpu.sync_copy(htab_hbm, htab_v)

    lane = lax.iota(jnp.int32, L)

    def body(g, _):
        obase = (g * L + lane) * D_OUT
        usrc = (uid_v[pl.ds(g * L, L)] + 1) * D_USER
        for j in range(D_USER):
            vals = plsc.load_gather(utab_v, [usrc + j])
            plsc.store_scatter(outbuf_v, [obase + j], vals)
        d = dow_v[pl.ds(g * L, L)]
        for c in range(D_DOW):
            v = jnp.where(d == c, 1.0, 0.0).astype(jnp.float32)
            plsc.store_scatter(outbuf_v, [obase + (D_USER + c)], v)
        hsrc = jnp.minimum(hod_v[pl.ds(g * L, L)] + 1, 23) * D_HOD
        for j in range(D_HOD):
            vals = plsc.load_gather(htab_v, [hsrc + j])
            plsc.store_scatter(outbuf_v, [obase + (D_USER + D_DOW + j)], vals)
        return 0

    lax.fori_loop(0, GROUPS, body, 0)

    pltpu.sync_copy(outbuf_v, out_hbm.at[pl.ds(base * D_OUT, B_PER_W * D_OUT)])


@jax.jit
def kernel(user_id, dow, hod, user_table, hod_table):
    mesh = plsc.VectorSubcoreMesh(core_axis_name="c", subcore_axis_name="s")
    run = functools.partial(
        pl.kernel, mesh=mesh,
        compiler_params=pltpu.CompilerParams(needs_layout_passes=False),
        out_type=jax.ShapeDtypeStruct((BATCH * D_OUT,), jnp.float32),
        scratch_types=[
            pltpu.VMEM((B_PER_W,), jnp.int32),
            pltpu.VMEM((B_PER_W,), jnp.int32),
            pltpu.VMEM((B_PER_W,), jnp.int32),
            pltpu.VMEM(((VOCAB + 1) * D_USER,), jnp.float32),
            pltpu.VMEM((24 * D_HOD,), jnp.float32),
            pltpu.VMEM((B_PER_W * D_OUT,), jnp.float32),
        ],
    )(_sc_kernel)
    out_flat = run(user_id, dow, hod, user_table.reshape(-1),
                   hod_table.reshape(-1))
    return out_flat.reshape(BATCH, D_OUT)


# speedup vs baseline: 2.2734x; 1.0042x over previous
import functools
import jax
import jax.numpy as jnp
from jax import lax
from jax.experimental import pallas as pl
from jax.experimental.pallas import tpu as pltpu
from jax.experimental.pallas import tpu_sc as plsc

BATCH = 16384
VOCAB = 1000
D_USER = 32
D_DOW = 7
D_HOD = 4
D_OUT = D_USER + D_DOW + D_HOD

NC = 2
NS = 16
NW = NC * NS
L = 16
B_PER_W = BATCH // NW
GROUPS = B_PER_W // L


def _sc_kernel(user_id_hbm, dow_hbm, hod_hbm, utab_hbm, htab_hbm, out_hbm,
               uid_v, dow_v, hod_v, utab_v, htab_v, outbuf_v):
    wid = lax.axis_index("s") * NC + lax.axis_index("c")
    base = wid * B_PER_W

    pltpu.sync_copy(user_id_hbm.at[pl.ds(base, B_PER_W)], uid_v)
    pltpu.sync_copy(dow_hbm.at[pl.ds(base, B_PER_W)], dow_v)
    pltpu.sync_copy(hod_hbm.at[pl.ds(base, B_PER_W)], hod_v)
    pltpu.sync_copy(utab_hbm, utab_v)
    pltpu.sync_copy(htab_hbm, htab_v)

    lane = lax.iota(jnp.int32, L)

    @plsc.parallel_loop(0, GROUPS, unroll=4)
    def _group(g):
        obase = (g * L + lane) * D_OUT
        usrc = (uid_v[pl.ds(g * L, L)] + 1) * D_USER
        for j in range(D_USER):
            vals = plsc.load_gather(utab_v, [usrc + j])
            plsc.store_scatter(outbuf_v, [obase + j], vals)
        d = dow_v[pl.ds(g * L, L)]
        for c in range(D_DOW):
            v = jnp.where(d == c, 1.0, 0.0).astype(jnp.float32)
            plsc.store_scatter(outbuf_v, [obase + (D_USER + c)], v)
        hsrc = jnp.minimum(hod_v[pl.ds(g * L, L)] + 1, 23) * D_HOD
        for j in range(D_HOD):
            vals = plsc.load_gather(htab_v, [hsrc + j])
            plsc.store_scatter(outbuf_v, [obase + (D_USER + D_DOW + j)], vals)

    pltpu.sync_copy(outbuf_v, out_hbm.at[pl.ds(base * D_OUT, B_PER_W * D_OUT)])


@jax.jit
def kernel(user_id, dow, hod, user_table, hod_table):
    mesh = plsc.VectorSubcoreMesh(core_axis_name="c", subcore_axis_name="s")
    run = functools.partial(
        pl.kernel, mesh=mesh,
        compiler_params=pltpu.CompilerParams(needs_layout_passes=False),
        out_type=jax.ShapeDtypeStruct((BATCH * D_OUT,), jnp.float32),
        scratch_types=[
            pltpu.VMEM((B_PER_W,), jnp.int32),
            pltpu.VMEM((B_PER_W,), jnp.int32),
            pltpu.VMEM((B_PER_W,), jnp.int32),
            pltpu.VMEM(((VOCAB + 1) * D_USER,), jnp.float32),
            pltpu.VMEM((24 * D_HOD,), jnp.float32),
            pltpu.VMEM((B_PER_W * D_OUT,), jnp.float32),
        ],
    )(_sc_kernel)
    out_flat = run(user_id, dow, hod, user_table.reshape(-1),
                   hod_table.reshape(-1))
    return out_flat.reshape(BATCH, D_OUT)


# speedup vs baseline: 2.5667x; 1.1290x over previous
import functools
import jax
import jax.numpy as jnp
from jax import lax
from jax.experimental import pallas as pl
from jax.experimental.pallas import tpu as pltpu
from jax.experimental.pallas import tpu_sc as plsc

BATCH = 16384
VOCAB = 1000
D_USER = 32
D_DOW = 7
D_HOD = 4
D_OUT = D_USER + D_DOW + D_HOD

NC = 2
NS = 16
NW = NC * NS
L = 16
B_PER_W = BATCH // NW
GROUPS = B_PER_W // L


def _sc_kernel(user_id_hbm, dow_hbm, hod_hbm, utab_hbm, htab_hbm, out_hbm,
               uid_v, dow_v, hod_v, utab_v, htab_v, outbuf_v):
    wid = lax.axis_index("s") * NC + lax.axis_index("c")
    base = wid * B_PER_W

    pltpu.sync_copy(user_id_hbm.at[pl.ds(base, B_PER_W)], uid_v)
    pltpu.sync_copy(dow_hbm.at[pl.ds(base, B_PER_W)], dow_v)
    pltpu.sync_copy(hod_hbm.at[pl.ds(base, B_PER_W)], hod_v)
    pltpu.sync_copy(utab_hbm, utab_v)
    pltpu.sync_copy(htab_hbm, htab_v)

    lane = lax.iota(jnp.int32, L)

    @plsc.parallel_loop(0, GROUPS, unroll=4)
    def _group(g):
        rows = g * L + lane
        usrc = (uid_v[pl.ds(g * L, L)] + 1) * D_USER
        for j in range(D_USER):
            col = jnp.full((L,), j, jnp.int32)
            vals = plsc.load_gather(utab_v, [usrc + j])
            plsc.store_scatter(outbuf_v, [rows, col], vals)
        d = dow_v[pl.ds(g * L, L)]
        for c in range(D_DOW):
            v = jnp.where(d == c, 1.0, 0.0).astype(jnp.float32)
            col = jnp.full((L,), D_USER + c, jnp.int32)
            plsc.store_scatter(outbuf_v, [rows, col], v)
        hsrc = jnp.minimum(hod_v[pl.ds(g * L, L)] + 1, 23) * D_HOD
        for j in range(D_HOD):
            vals = plsc.load_gather(htab_v, [hsrc + j])
            ocol = jnp.full((L,), D_USER + D_DOW + j, jnp.int32)
            plsc.store_scatter(outbuf_v, [rows, ocol], vals)

    pltpu.sync_copy(outbuf_v, out_hbm.at[pl.ds(base, B_PER_W)])


@jax.jit
def kernel(user_id, dow, hod, user_table, hod_table):
    mesh = plsc.VectorSubcoreMesh(core_axis_name="c", subcore_axis_name="s")
    run = functools.partial(
        pl.kernel, mesh=mesh,
        compiler_params=pltpu.CompilerParams(needs_layout_passes=False),
        out_type=jax.ShapeDtypeStruct((BATCH, D_OUT), jnp.float32),
        scratch_types=[
            pltpu.VMEM((B_PER_W,), jnp.int32),
            pltpu.VMEM((B_PER_W,), jnp.int32),
            pltpu.VMEM((B_PER_W,), jnp.int32),
            pltpu.VMEM(((VOCAB + 1) * D_USER,), jnp.float32),
            pltpu.VMEM((24 * D_HOD,), jnp.float32),
            pltpu.VMEM((B_PER_W, D_OUT), jnp.float32),
        ],
    )(_sc_kernel)
    return run(user_id, dow, hod, user_table.reshape(-1),
               hod_table.reshape(-1))


# speedup vs baseline: 2.6602x; 1.0364x over previous
import functools
import jax
import jax.numpy as jnp
from jax import lax
from jax.experimental import pallas as pl
from jax.experimental.pallas import tpu as pltpu
from jax.experimental.pallas import tpu_sc as plsc

BATCH = 16384
VOCAB = 1000
D_USER = 32
U_STRIDE = 33
D_DOW = 7
D_HOD = 4
H_STRIDE = 5
D_OUT = D_USER + D_DOW + D_HOD

NC = 2
NS = 16
NW = NC * NS
L = 16
B_PER_W = BATCH // NW
GROUPS = B_PER_W // L


def _sc_kernel(user_id_hbm, dow_hbm, hod_hbm, utab_hbm, htab_hbm, out_hbm,
               uid_v, dow_v, hod_v, utab_v, htab_v, outbuf_v):
    wid = lax.axis_index("s") * NC + lax.axis_index("c")
    base = wid * B_PER_W

    pltpu.sync_copy(user_id_hbm.at[pl.ds(base, B_PER_W)], uid_v)
    pltpu.sync_copy(dow_hbm.at[pl.ds(base, B_PER_W)], dow_v)
    pltpu.sync_copy(hod_hbm.at[pl.ds(base, B_PER_W)], hod_v)
    pltpu.sync_copy(utab_hbm, utab_v)
    pltpu.sync_copy(htab_hbm, htab_v)

    lane = lax.iota(jnp.int32, L)

    @plsc.parallel_loop(0, GROUPS, unroll=4)
    def _group(g):
        rows = g * L + lane
        usrc = (uid_v[pl.ds(g * L, L)] + 1) * U_STRIDE
        for j in range(D_USER):
            col = jnp.full((L,), j, jnp.int32)
            vals = plsc.load_gather(utab_v, [usrc + j])
            plsc.store_scatter(outbuf_v, [rows, col], vals)
        d = dow_v[pl.ds(g * L, L)]
        for c in range(D_DOW):
            v = jnp.where(d == c, 1.0, 0.0).astype(jnp.float32)
            col = jnp.full((L,), D_USER + c, jnp.int32)
            plsc.store_scatter(outbuf_v, [rows, col], v)
        hsrc = jnp.minimum(hod_v[pl.ds(g * L, L)] + 1, 23) * H_STRIDE
        for j in range(D_HOD):
            vals = plsc.load_gather(htab_v, [hsrc + j])
            ocol = jnp.full((L,), D_USER + D_DOW + j, jnp.int32)
            plsc.store_scatter(outbuf_v, [rows, ocol], vals)

    pltpu.sync_copy(outbuf_v, out_hbm.at[pl.ds(base, B_PER_W)])


@jax.jit
def kernel(user_id, dow, hod, user_table, hod_table):
    mesh = plsc.VectorSubcoreMesh(core_axis_name="c", subcore_axis_name="s")
    run = functools.partial(
        pl.kernel, mesh=mesh,
        compiler_params=pltpu.CompilerParams(
            needs_layout_passes=False, use_tc_tiling_on_sc=False),
        out_type=jax.ShapeDtypeStruct((BATCH, D_OUT), jnp.float32),
        scratch_types=[
            pltpu.VMEM((B_PER_W,), jnp.int32),
            pltpu.VMEM((B_PER_W,), jnp.int32),
            pltpu.VMEM((B_PER_W,), jnp.int32),
            pltpu.VMEM(((VOCAB + 1) * U_STRIDE,), jnp.float32),
            pltpu.VMEM((24 * H_STRIDE,), jnp.float32),
            pltpu.VMEM((B_PER_W, D_OUT), jnp.float32),
        ],
    )(_sc_kernel)
    utab_pad = jnp.pad(user_table, ((0, 0), (0, U_STRIDE - D_USER))).reshape(-1)
    htab_pad = jnp.pad(hod_table, ((0, 0), (0, H_STRIDE - D_HOD))).reshape(-1)
    return run(user_id, dow, hod, utab_pad, htab_pad)


# speedup vs baseline: 4.0098x; 1.5073x over previous
import functools
import jax
import jax.numpy as jnp
from jax import lax
from jax.experimental import pallas as pl
from jax.experimental.pallas import tpu as pltpu
from jax.experimental.pallas import tpu_sc as plsc

BATCH = 16384
VOCAB = 1000
D_USER = 32
D_DOW = 7
D_HOD = 4
D_OUT = D_USER + D_DOW + D_HOD

NC = 2
NS = 16
NW = NC * NS
L = 16
B_PER_W = BATCH // NW
GROUPS = B_PER_W // L


def _sc_kernel(user_id_hbm, dow_hbm, hod_hbm, utab_hbm, htab_hbm, out_hbm,
               uid_v, dow_v, hod_v, utab_v, htab_v, outbuf_v):
    wid = lax.axis_index("s") * NC + lax.axis_index("c")
    base = wid * B_PER_W

    pltpu.sync_copy(user_id_hbm.at[pl.ds(base, B_PER_W)], uid_v)
    pltpu.sync_copy(dow_hbm.at[pl.ds(base, B_PER_W)], dow_v)
    pltpu.sync_copy(hod_hbm.at[pl.ds(base, B_PER_W)], hod_v)
    pltpu.sync_copy(utab_hbm, utab_v)
    pltpu.sync_copy(htab_hbm, htab_v)

    lane = lax.iota(jnp.int32, L)

    @plsc.parallel_loop(0, GROUPS, unroll=4)
    def _group(g):
        cols = g * L + lane
        uidx = uid_v[pl.ds(g * L, L)] + 1
        for j in range(D_USER):
            row = jnp.full((L,), j, jnp.int32)
            vals = plsc.load_gather(utab_v, [row, uidx])
            plsc.store_scatter(outbuf_v, [row, cols], vals)
        d = dow_v[pl.ds(g * L, L)]
        for c in range(D_DOW):
            v = jnp.where(d == c, 1.0, 0.0).astype(jnp.float32)
            row = jnp.full((L,), D_USER + c, jnp.int32)
            plsc.store_scatter(outbuf_v, [row, cols], v)
        hidx = jnp.minimum(hod_v[pl.ds(g * L, L)] + 1, 23)
        for j in range(D_HOD):
            row = jnp.full((L,), j, jnp.int32)
            vals = plsc.load_gather(htab_v, [row, hidx])
            orow = jnp.full((L,), D_USER + D_DOW + j, jnp.int32)
            plsc.store_scatter(outbuf_v, [orow, cols], vals)

    pltpu.sync_copy(outbuf_v, out_hbm.at[:, pl.ds(base, B_PER_W)])


@jax.jit
def kernel(user_id, dow, hod, user_table, hod_table):
    mesh = plsc.VectorSubcoreMesh(core_axis_name="c", subcore_axis_name="s")
    run = functools.partial(
        pl.kernel, mesh=mesh,
        compiler_params=pltpu.CompilerParams(needs_layout_passes=False),
        out_type=jax.ShapeDtypeStruct((D_OUT, BATCH), jnp.float32),
        scratch_types=[
            pltpu.VMEM((B_PER_W,), jnp.int32),
            pltpu.VMEM((B_PER_W,), jnp.int32),
            pltpu.VMEM((B_PER_W,), jnp.int32),
            pltpu.VMEM((D_USER, VOCAB + 1), jnp.float32),
            pltpu.VMEM((D_HOD, 24), jnp.float32),
            pltpu.VMEM((D_OUT, B_PER_W), jnp.float32),
        ],
    )(_sc_kernel)
    out_t = run(user_id, dow, hod, user_table.T, hod_table.T)
    return out_t.T


# speedup vs baseline: 4.8654x; 1.2134x over previous
import functools
import jax
import jax.numpy as jnp
from jax import lax
from jax.experimental import pallas as pl
from jax.experimental.pallas import tpu as pltpu
from jax.experimental.pallas import tpu_sc as plsc

BATCH = 16384
VOCAB = 1000
D_USER = 32
D_DOW = 7
D_HOD = 4
D_OUT = D_USER + D_DOW + D_HOD

NC = 2
NS = 16
NW = NC * NS
L = 16
B_PER_W = BATCH // NW
GROUPS = B_PER_W // L


def _sc_kernel(user_id_hbm, dow_hbm, hod_hbm, utab_hbm, htab_hbm, out_hbm,
               uid_v, dow_v, hod_v, utab_v, htab_v, outbuf_v, sem):
    wid = lax.axis_index("s") * NC + lax.axis_index("c")
    base = wid * B_PER_W

    utab_cp = pltpu.async_copy(utab_hbm, utab_v, sem)

    pltpu.sync_copy(user_id_hbm.at[pl.ds(base, B_PER_W)], uid_v)
    pltpu.sync_copy(dow_hbm.at[pl.ds(base, B_PER_W)], dow_v)
    pltpu.sync_copy(hod_hbm.at[pl.ds(base, B_PER_W)], hod_v)
    pltpu.sync_copy(htab_hbm, htab_v)

    lane = lax.iota(jnp.int32, L)
    zeros = jnp.zeros((L,), jnp.float32)
    ones = jnp.ones((L,), jnp.float32)

    @plsc.parallel_loop(0, GROUPS, unroll=4)
    def _prep(g):
        for c in range(D_DOW):
            outbuf_v[D_USER + c, pl.ds(g * L, L)] = zeros
        cols = g * L + lane
        d = dow_v[pl.ds(g * L, L)]
        plsc.store_scatter(outbuf_v, [D_USER + d, cols], ones)
        hsrc = jnp.minimum(hod_v[pl.ds(g * L, L)] + 1, 23)
        for j in range(D_HOD):
            vals = plsc.load_gather(htab_v, [j * 24 + hsrc])
            outbuf_v[D_USER + D_DOW + j, pl.ds(g * L, L)] = vals

    utab_cp.wait()

    @plsc.parallel_loop(0, GROUPS, unroll=4)
    def _user(g):
        uidx = uid_v[pl.ds(g * L, L)] + 1
        for j in range(D_USER):
            vals = plsc.load_gather(utab_v, [j * (VOCAB + 1) + uidx])
            outbuf_v[j, pl.ds(g * L, L)] = vals

    pltpu.sync_copy(outbuf_v, out_hbm.at[:, pl.ds(base, B_PER_W)])


@jax.jit
def kernel(user_id, dow, hod, user_table, hod_table):
    mesh = plsc.VectorSubcoreMesh(core_axis_name="c", subcore_axis_name="s")
    run = functools.partial(
        pl.kernel, mesh=mesh,
        compiler_params=pltpu.CompilerParams(needs_layout_passes=False),
        out_type=jax.ShapeDtypeStruct((D_OUT, BATCH), jnp.float32),
        scratch_types=[
            pltpu.VMEM((B_PER_W,), jnp.int32),
            pltpu.VMEM((B_PER_W,), jnp.int32),
            pltpu.VMEM((B_PER_W,), jnp.int32),
            pltpu.VMEM((D_USER * (VOCAB + 1),), jnp.float32),
            pltpu.VMEM((D_HOD * 24,), jnp.float32),
            pltpu.VMEM((D_OUT, B_PER_W), jnp.float32),
            pltpu.SemaphoreType.DMA,
        ],
    )(_sc_kernel)
    out_t = run(user_id, dow, hod, user_table.T.reshape(-1),
                hod_table.T.reshape(-1))
    return out_t.T


# speedup vs baseline: 4.8710x; 1.0011x over previous
import functools
import jax
import jax.numpy as jnp
from jax import lax
from jax.experimental import pallas as pl
from jax.experimental.pallas import tpu as pltpu
from jax.experimental.pallas import tpu_sc as plsc

BATCH = 16384
VOCAB = 1000
D_USER = 32
D_DOW = 7
D_HOD = 4
D_OUT = D_USER + D_DOW + D_HOD

NC = 2
NS = 16
NW = NC * NS
L = 16
B_PER_W = BATCH // NW
GROUPS = B_PER_W // L


def _sc_kernel(user_id_hbm, dow_hbm, hod_hbm, utab_hbm, htab_hbm, out_hbm,
               uid_v, dow_v, hod_v, utab_v, htab_v, outbuf_v, sem):
    wid = lax.axis_index("s") * NC + lax.axis_index("c")
    base = wid * B_PER_W

    utab_cp = pltpu.async_copy(utab_hbm, utab_v, sem)

    pltpu.sync_copy(user_id_hbm.at[pl.ds(base, B_PER_W)], uid_v)
    pltpu.sync_copy(dow_hbm.at[pl.ds(base, B_PER_W)], dow_v)
    pltpu.sync_copy(hod_hbm.at[pl.ds(base, B_PER_W)], hod_v)
    pltpu.sync_copy(htab_hbm, htab_v)

    lane = lax.iota(jnp.int32, L)
    zeros = jnp.zeros((L,), jnp.float32)
    ones = jnp.ones((L,), jnp.float32)

    @plsc.parallel_loop(0, GROUPS, unroll=4)
    def _prep(g):
        for c in range(D_DOW):
            outbuf_v[D_USER + c, pl.ds(g * L, L)] = zeros
        cols = g * L + lane
        d = dow_v[pl.ds(g * L, L)]
        plsc.store_scatter(outbuf_v, [D_USER + d, cols], ones)
        hsrc = jnp.minimum(hod_v[pl.ds(g * L, L)] + 1, 23)
        for j in range(D_HOD):
            vals = plsc.load_gather(htab_v, [j * 24 + hsrc])
            outbuf_v[D_USER + D_DOW + j, pl.ds(g * L, L)] = vals

    utab_cp.wait()

    @plsc.parallel_loop(0, GROUPS, unroll=4)
    def _user(g):
        uidx = uid_v[pl.ds(g * L, L)] + 1
        for j in range(D_USER):
            vals = plsc.load_gather(utab_v, [j * (VOCAB + 1) + uidx])
            outbuf_v[j, pl.ds(g * L, L)] = vals

    pltpu.sync_copy(outbuf_v, out_hbm.at[:, pl.ds(base, B_PER_W)])


@jax.jit
def kernel(user_id, dow, hod, user_table, hod_table):
    mesh = plsc.VectorSubcoreMesh(core_axis_name="c", subcore_axis_name="s")
    run = functools.partial(
        pl.kernel, mesh=mesh,
        compiler_params=pltpu.CompilerParams(
            needs_layout_passes=False, skip_device_barrier=True),
        out_type=jax.ShapeDtypeStruct((D_OUT, BATCH), jnp.float32),
        scratch_types=[
            pltpu.VMEM((B_PER_W,), jnp.int32),
            pltpu.VMEM((B_PER_W,), jnp.int32),
            pltpu.VMEM((B_PER_W,), jnp.int32),
            pltpu.VMEM((D_USER * (VOCAB + 1),), jnp.float32),
            pltpu.VMEM((D_HOD * 24,), jnp.float32),
            pltpu.VMEM((D_OUT, B_PER_W), jnp.float32),
            pltpu.SemaphoreType.DMA,
        ],
    )(_sc_kernel)
    out_t = run(user_id, dow, hod, user_table.T.reshape(-1),
                hod_table.T.reshape(-1))
    return out_t.T
